# baseline (speedup 1.0000x reference)
import jax
import jax.numpy as jnp
from jax import lax
from jax.experimental import pallas as pl
from jax.experimental.pallas import tpu as pltpu
from jax.experimental.pallas import tpu_sc as plsc

MPRIME = (1 << 31) - 1
N_ROWS = 1000000
EMB_DIM = 32
CHUNKS = 8
B_TOTAL = 16384
NC = 2
NS = 16
NW = NC * NS
B_PER_W = B_TOTAL // NW
SUB = 128
N_SUB = B_PER_W // SUB


def _mersenne_hash(x0, x1, a_lo, a_hi, b_add, out_range):
    m = jnp.uint32(MPRIME)
    p0 = x0 * a_lo
    pm = x0 * a_hi + x1 * a_lo
    p2 = x1 * a_hi
    ra = (p0 >> 31) + (p0 & m)
    ra = (ra >> 31) + (ra & m)
    rb = (pm >> 15) + ((pm & jnp.uint32(0x7FFF)) << 16)
    rb = (rb >> 31) + (rb & m)
    s = ra + rb
    s = (s >> 31) + (s & m)
    s = s + (p2 << 1)
    s = (s >> 31) + (s & m)
    s = s + b_add
    s = (s >> 31) + (s & m)
    s = (s >> 31) + (s & m)
    s = jnp.where(s == m, jnp.uint32(0), s)
    return lax.rem(plsc.bitcast(s, jnp.int32), jnp.int32(out_range))


def _remap_row(i):
    return ((i & jnp.int32(~(TBLK - 1))) | ((i & jnp.int32(TQ - 1)) << 2)
            | ((i >> 12) & jnp.int32(3)))


def _fori(n, body):
    lax.fori_loop(jnp.int32(0), jnp.int32(n), body, jnp.int32(0))


def _body(x_hbm, table_hbm, w_hbm, coef_hbm, out_hbm,
          xv, coef_v, idx0_v, idx1_v, rows_v, wv_v, out_v, sem):
    wid = lax.axis_index("s") * NC + lax.axis_index("c")
    base = wid * B_PER_W
    pltpu.sync_copy(x_hbm.at[pl.ds(base, B_PER_W)], xv)
    pltpu.sync_copy(coef_hbm, coef_v)

    c01 = coef_v[pl.ds(0, 16)]
    c23 = coef_v[pl.ds(16, 16)]
    cparams = []
    for c in range(CHUNKS):
        a0, b0, a1, b1 = c01[c], c01[8 + c], c23[c], c23[8 + c]
        cparams.append((a0 & jnp.uint32(0xFFFF), a0 >> 16, b0,
                        a1 & jnp.uint32(0xFFFF), a1 >> 16, b1))

    def sblock(s, carry):
      with jax.named_scope("hash"):
        for c in range(CHUNKS):
            a0_lo, a0_hi, b0, a1_lo, a1_hi, b1 = cparams[c]

            def vbody(v, _, c=c, a0_lo=a0_lo, a0_hi=a0_hi, b0=b0,
                      a1_lo=a1_lo, a1_hi=a1_hi, b1=b1):
                xu = plsc.bitcast(xv[pl.ds(s * SUB + v * 16, 16)], jnp.uint32)
                x0 = xu & jnp.uint32(0xFFFF)
                x1 = xu >> 16
                idx0_v[c, pl.ds(v * 16, 16)] = _remap_row(_mersenne_hash(
                    x0, x1, a0_lo, a0_hi, b0, N_ROWS))
                idx1_v[c, pl.ds(v * 16, 16)] = _mersenne_hash(
                    x0, x1, a1_lo, a1_hi, b1, N_ROWS * EMB_DIM)
                return _

            _fori(SUB // 16, vbody)

      with jax.named_scope("gather"):
        copies = []
        for c in range(CHUNKS):
            ci = jnp.int32(c)
            copies.append(pltpu.make_async_copy(
                table_hbm.at[idx0_v.at[ci]], rows_v.at[ci], sem))
            copies.append(pltpu.make_async_copy(
                w_hbm.at[idx1_v.at[ci]], wv_v.at[ci], sem))
        for cp in copies:
            cp.start()
        for cp in copies:
            cp.wait()

      with jax.named_scope("acc"):
        def racc(g, _):
            r0 = g * 16
            wvecs = [wv_v[c, pl.ds(r0, 16)] for c in range(CHUNKS)]
            for j in range(16):
                acc0 = jnp.zeros((16,), jnp.float32)
                acc1 = jnp.zeros((16,), jnp.float32)
                for c in range(CHUNKS):
                    w = wvecs[c][j]
                    acc0 = acc0 + rows_v[c, r0 + j, pl.ds(0, 16)] * w
                    acc1 = acc1 + rows_v[c, r0 + j, pl.ds(16, 16)] * w
                out_v[pl.ds((r0 + j) * EMB_DIM, 16)] = acc0 * 0.125
                out_v[pl.ds((r0 + j) * EMB_DIM + 16, 16)] = acc1 * 0.125
            return _

        _fori(SUB // 16, racc)
        pltpu.sync_copy(
            out_v, out_hbm.at[pl.ds((base + s * SUB) * EMB_DIM, SUB * EMB_DIM)])
        return carry

    _fori(N_SUB, sblock)


TBLK = 16384
TQ = TBLK // 4
TGRID = (N_ROWS + TBLK - 1) // TBLK
ROWS_PAD = TGRID * TBLK


def _transpose_body(tt_ref, out_ref):
    x = tt_ref[...]
    x4 = jnp.concatenate(
        [x[:, k * TQ:(k + 1) * TQ] for k in range(4)], axis=0)
    out_ref[...] = x4.T


def _relayout_table(table_t):
    return pl.pallas_call(
        _transpose_body,
        grid=(TGRID,),
        in_specs=[pl.BlockSpec((32, TBLK), lambda i: (jnp.int32(0), i))],
        out_specs=pl.BlockSpec((TQ, 128), lambda i: (i, jnp.int32(0))),
        out_shape=jax.ShapeDtypeStruct((TGRID * TQ, 128), jnp.float32),
    )(table_t)


@jax.jit
def _run(x_i32, table, w_flat, coef):
    mesh = plsc.VectorSubcoreMesh(core_axis_name="c", subcore_axis_name="s")
    return pl.kernel(
        _body,
        out_type=jax.ShapeDtypeStruct((B_TOTAL * EMB_DIM,), jnp.float32),
        mesh=mesh,
        compiler_params=pltpu.CompilerParams(use_tc_tiling_on_sc=False),
        scratch_types=[
            pltpu.VMEM((B_PER_W,), jnp.int32),
            pltpu.VMEM((2 * 16,), jnp.uint32),
            pltpu.VMEM((CHUNKS, SUB), jnp.int32),
            pltpu.VMEM((CHUNKS, SUB), jnp.int32),
            pltpu.VMEM((CHUNKS, SUB, EMB_DIM), jnp.float32),
            pltpu.VMEM((CHUNKS, SUB), jnp.float32),
            pltpu.VMEM((SUB * EMB_DIM,), jnp.float32),
            pltpu.SemaphoreType.DMA,
        ],
    )(x_i32, table, w_flat, coef)


@jax.jit
def _entry(x, table, weights, h0_coeffs, h1_coeffs):
    x_i32 = x.astype(jnp.int32)
    w_flat = weights.reshape(-1)
    coef = jnp.concatenate([h0_coeffs[:, 0], h0_coeffs[:, 1],
                            h1_coeffs[:, 0], h1_coeffs[:, 1]]).astype(jnp.uint32)
    t_lin = _relayout_table(table.T).reshape(ROWS_PAD, EMB_DIM)
    return _run(x_i32, t_lin, w_flat, coef).reshape(B_TOTAL, EMB_DIM)


def kernel(x, table, weights, h0_coeffs, h1_coeffs):
    return _entry(x, table, weights, h0_coeffs, h1_coeffs)


# speedup vs baseline: 1.0785x; 1.0785x over previous
import jax
import jax.numpy as jnp
from jax import lax
from jax.experimental import pallas as pl
from jax.experimental.pallas import tpu as pltpu
from jax.experimental.pallas import tpu_sc as plsc

MPRIME = (1 << 31) - 1
N_ROWS = 1000000
EMB_DIM = 32
CHUNKS = 8
B_TOTAL = 16384
NC = 2
NS = 16
NW = NC * NS
B_PER_W = B_TOTAL // NW
SUB = 128
N_SUB = B_PER_W // SUB


RMAGIC = 1125899907


def _magic_rem(s, shift, divisor):
    m0 = jnp.uint32(RMAGIC & 0xFFFF)
    m1 = jnp.uint32(RMAGIC >> 16)
    n0 = s & jnp.uint32(0xFFFF)
    n1 = s >> 16
    lo = n0 * m0
    mid = n0 * m1 + n1 * m0
    u = mid + (lo >> 16)
    hi = n1 * m1 + (u >> 16)
    q = hi >> shift
    return s - q * jnp.uint32(divisor)


def _mersenne_hash(x0, x1, a_lo, a_hi, b_add, out_range):
    m = jnp.uint32(MPRIME)
    p0 = x0 * a_lo
    pm = x0 * a_hi + x1 * a_lo
    p2 = x1 * a_hi
    ra = (p0 >> 31) + (p0 & m)
    ra = (ra >> 31) + (ra & m)
    rb = (pm >> 15) + ((pm & jnp.uint32(0x7FFF)) << 16)
    rb = (rb >> 31) + (rb & m)
    s = ra + rb
    s = (s >> 31) + (s & m)
    s = s + (p2 << 1)
    s = (s >> 31) + (s & m)
    s = s + b_add
    s = (s >> 31) + (s & m)
    s = (s >> 31) + (s & m)
    s = jnp.where(s == m, jnp.uint32(0), s)
    shift = 18 if out_range == N_ROWS else 23
    return plsc.bitcast(_magic_rem(s, shift, out_range), jnp.int32)


def _remap_row(i):
    return ((i & jnp.int32(~(TBLK - 1))) | ((i & jnp.int32(TQ - 1)) << 2)
            | ((i >> 12) & jnp.int32(3)))


def _fori(n, body):
    lax.fori_loop(jnp.int32(0), jnp.int32(n), body, jnp.int32(0))


def _body(x_hbm, table_hbm, w_hbm, coef_hbm, out_hbm,
          xv, coef_v, idx0_v, idx1_v, rows_v, wv_v, out_v, sem):
    wid = lax.axis_index("s") * NC + lax.axis_index("c")
    base = wid * B_PER_W
    pltpu.sync_copy(x_hbm.at[pl.ds(base, B_PER_W)], xv)
    pltpu.sync_copy(coef_hbm, coef_v)

    c01 = coef_v[pl.ds(0, 16)]
    c23 = coef_v[pl.ds(16, 16)]
    cparams = []
    for c in range(CHUNKS):
        a0, b0, a1, b1 = c01[c], c01[8 + c], c23[c], c23[8 + c]
        cparams.append((a0 & jnp.uint32(0xFFFF), a0 >> 16, b0,
                        a1 & jnp.uint32(0xFFFF), a1 >> 16, b1))

    def sblock(s, carry):
      with jax.named_scope("hash"):
        for c in range(CHUNKS):
            a0_lo, a0_hi, b0, a1_lo, a1_hi, b1 = cparams[c]

            def vbody(v, _, c=c, a0_lo=a0_lo, a0_hi=a0_hi, b0=b0,
                      a1_lo=a1_lo, a1_hi=a1_hi, b1=b1):
                for u in range(2):
                    off = s * SUB + v * 32 + u * 16
                    xu = plsc.bitcast(xv[pl.ds(off, 16)], jnp.uint32)
                    x0 = xu & jnp.uint32(0xFFFF)
                    x1 = xu >> 16
                    idx0_v[c, pl.ds(v * 32 + u * 16, 16)] = _remap_row(
                        _mersenne_hash(x0, x1, a0_lo, a0_hi, b0, N_ROWS))
                    idx1_v[c, pl.ds(v * 32 + u * 16, 16)] = _mersenne_hash(
                        x0, x1, a1_lo, a1_hi, b1, N_ROWS * EMB_DIM)
                return _

            _fori(SUB // 32, vbody)

      with jax.named_scope("gather"):
        copies = []
        for c in range(CHUNKS):
            ci = jnp.int32(c)
            copies.append(pltpu.make_async_copy(
                table_hbm.at[idx0_v.at[ci]], rows_v.at[ci], sem))
            copies.append(pltpu.make_async_copy(
                w_hbm.at[idx1_v.at[ci]], wv_v.at[ci], sem))
        for cp in copies:
            cp.start()
        for cp in copies:
            cp.wait()

      with jax.named_scope("acc"):
        def racc(g, _):
            r0 = g * 16
            wvecs = [wv_v[c, pl.ds(r0, 16)] for c in range(CHUNKS)]
            for j in range(16):
                acc0 = jnp.zeros((16,), jnp.float32)
                acc1 = jnp.zeros((16,), jnp.float32)
                for c in range(CHUNKS):
                    w = wvecs[c][j]
                    acc0 = acc0 + rows_v[c, r0 + j, pl.ds(0, 16)] * w
                    acc1 = acc1 + rows_v[c, r0 + j, pl.ds(16, 16)] * w
                out_v[pl.ds((r0 + j) * EMB_DIM, 16)] = acc0 * 0.125
                out_v[pl.ds((r0 + j) * EMB_DIM + 16, 16)] = acc1 * 0.125
            return _

        _fori(SUB // 16, racc)
        pltpu.sync_copy(
            out_v, out_hbm.at[pl.ds((base + s * SUB) * EMB_DIM, SUB * EMB_DIM)])
        return carry

    _fori(N_SUB, sblock)


TBLK = 16384
TQ = TBLK // 4
TGRID = (N_ROWS + TBLK - 1) // TBLK
ROWS_PAD = TGRID * TBLK


def _transpose_body(tt_ref, out_ref):
    x = tt_ref[...]
    x4 = jnp.concatenate(
        [x[:, k * TQ:(k + 1) * TQ] for k in range(4)], axis=0)
    out_ref[...] = x4.T


def _relayout_table(table_t):
    return pl.pallas_call(
        _transpose_body,
        grid=(TGRID,),
        in_specs=[pl.BlockSpec((32, TBLK), lambda i: (jnp.int32(0), i))],
        out_specs=pl.BlockSpec((TQ, 128), lambda i: (i, jnp.int32(0))),
        out_shape=jax.ShapeDtypeStruct((TGRID * TQ, 128), jnp.float32),
    )(table_t)


@jax.jit
def _run(x_i32, table, w_flat, coef):
    mesh = plsc.VectorSubcoreMesh(core_axis_name="c", subcore_axis_name="s")
    return pl.kernel(
        _body,
        out_type=jax.ShapeDtypeStruct((B_TOTAL * EMB_DIM,), jnp.float32),
        mesh=mesh,
        compiler_params=pltpu.CompilerParams(use_tc_tiling_on_sc=False),
        scratch_types=[
            pltpu.VMEM((B_PER_W,), jnp.int32),
            pltpu.VMEM((2 * 16,), jnp.uint32),
            pltpu.VMEM((CHUNKS, SUB), jnp.int32),
            pltpu.VMEM((CHUNKS, SUB), jnp.int32),
            pltpu.VMEM((CHUNKS, SUB, EMB_DIM), jnp.float32),
            pltpu.VMEM((CHUNKS, SUB), jnp.float32),
            pltpu.VMEM((SUB * EMB_DIM,), jnp.float32),
            pltpu.SemaphoreType.DMA,
        ],
    )(x_i32, table, w_flat, coef)


@jax.jit
def _entry(x, table, weights, h0_coeffs, h1_coeffs):
    x_i32 = x.astype(jnp.int32)
    w_flat = weights.reshape(-1)
    coef = jnp.concatenate([h0_coeffs[:, 0], h0_coeffs[:, 1],
                            h1_coeffs[:, 0], h1_coeffs[:, 1]]).astype(jnp.uint32)
    t_lin = _relayout_table(table.T).reshape(ROWS_PAD, EMB_DIM)
    return _run(x_i32, t_lin, w_flat, coef).reshape(B_TOTAL, EMB_DIM)


def kernel(x, table, weights, h0_coeffs, h1_coeffs):
    return _entry(x, table, weights, h0_coeffs, h1_coeffs)


# speedup vs baseline: 1.1825x; 1.0964x over previous
import jax
import jax.numpy as jnp
from jax import lax
from jax.experimental import pallas as pl
from jax.experimental.pallas import tpu as pltpu
from jax.experimental.pallas import tpu_sc as plsc

MPRIME = (1 << 31) - 1
N_ROWS = 1000000
EMB_DIM = 32
CHUNKS = 8
B_TOTAL = 16384
NC = 2
NS = 16
NW = NC * NS
B_PER_W = B_TOTAL // NW
SUB = 128
N_SUB = B_PER_W // SUB


RMAGIC = 1125899907


def _magic_rem(s, shift, divisor):
    m0 = jnp.uint32(RMAGIC & 0xFFFF)
    m1 = jnp.uint32(RMAGIC >> 16)
    n0 = s & jnp.uint32(0xFFFF)
    n1 = s >> 16
    lo = n0 * m0
    mid = n0 * m1 + n1 * m0
    u = mid + (lo >> 16)
    hi = n1 * m1 + (u >> 16)
    q = hi >> shift
    return s - q * jnp.uint32(divisor)


def _mersenne_hash(x0, x1, a_lo, a_hi, b_add, out_range):
    m = jnp.uint32(MPRIME)
    p0 = x0 * a_lo
    pm = x0 * a_hi + x1 * a_lo
    p2 = x1 * a_hi
    ra = (p0 >> 31) + (p0 & m)
    ra = (ra >> 31) + (ra & m)
    rb = (pm >> 15) + ((pm & jnp.uint32(0x7FFF)) << 16)
    rb = (rb >> 31) + (rb & m)
    s = ra + rb
    s = (s >> 31) + (s & m)
    s = s + (p2 << 1)
    s = (s >> 31) + (s & m)
    s = s + b_add
    s = (s >> 31) + (s & m)
    s = (s >> 31) + (s & m)
    s = jnp.where(s == m, jnp.uint32(0), s)
    shift = 18 if out_range == N_ROWS else 23
    return plsc.bitcast(_magic_rem(s, shift, out_range), jnp.int32)


def _remap_row(i):
    return ((i & jnp.int32(~(TBLK - 1))) | ((i & jnp.int32(TQ - 1)) << 2)
            | ((i >> (TQ.bit_length() - 1)) & jnp.int32(3)))


def _fori(n, body):
    lax.fori_loop(jnp.int32(0), jnp.int32(n), body, jnp.int32(0))


def _body(x_hbm, table_hbm, w_hbm, coef_hbm, out_hbm,
          xv, coef_v, idx0_v, idx1_v, rows_v, wv_v, out_v, sem):
    wid = lax.axis_index("s") * NC + lax.axis_index("c")
    base = wid * B_PER_W
    pltpu.sync_copy(x_hbm.at[pl.ds(base, B_PER_W)], xv)
    pltpu.sync_copy(coef_hbm, coef_v)

    c01 = coef_v[pl.ds(0, 16)]
    c23 = coef_v[pl.ds(16, 16)]
    cparams = []
    for c in range(CHUNKS):
        a0, b0, a1, b1 = c01[c], c01[8 + c], c23[c], c23[8 + c]
        cparams.append((a0 & jnp.uint32(0xFFFF), a0 >> 16, b0,
                        a1 & jnp.uint32(0xFFFF), a1 >> 16, b1))

    def sblock(s, carry):
      with jax.named_scope("hash"):
        for c in range(CHUNKS):
            a0_lo, a0_hi, b0, a1_lo, a1_hi, b1 = cparams[c]

            def vbody(v, _, c=c, a0_lo=a0_lo, a0_hi=a0_hi, b0=b0,
                      a1_lo=a1_lo, a1_hi=a1_hi, b1=b1):
                for u in range(2):
                    off = s * SUB + v * 32 + u * 16
                    xu = plsc.bitcast(xv[pl.ds(off, 16)], jnp.uint32)
                    x0 = xu & jnp.uint32(0xFFFF)
                    x1 = xu >> 16
                    idx0_v[c, pl.ds(v * 32 + u * 16, 16)] = _remap_row(
                        _mersenne_hash(x0, x1, a0_lo, a0_hi, b0, N_ROWS))
                    idx1_v[c, pl.ds(v * 32 + u * 16, 16)] = _mersenne_hash(
                        x0, x1, a1_lo, a1_hi, b1, N_ROWS * EMB_DIM)
                return _

            _fori(SUB // 32, vbody)

      with jax.named_scope("gather"):
        copies = []
        for c in range(CHUNKS):
            ci = jnp.int32(c)
            copies.append(pltpu.make_async_copy(
                table_hbm.at[idx0_v.at[ci]], rows_v.at[ci], sem))
            copies.append(pltpu.make_async_copy(
                w_hbm.at[idx1_v.at[ci]], wv_v.at[ci], sem))
        for cp in copies:
            cp.start()
        for cp in copies:
            cp.wait()

      with jax.named_scope("acc"):
        def racc(g, _):
            r0 = g * 16
            wvecs = [wv_v[c, pl.ds(r0, 16)] for c in range(CHUNKS)]
            for j in range(16):
                acc0 = jnp.zeros((16,), jnp.float32)
                acc1 = jnp.zeros((16,), jnp.float32)
                for c in range(CHUNKS):
                    w = wvecs[c][j]
                    acc0 = acc0 + rows_v[c, r0 + j, pl.ds(0, 16)] * w
                    acc1 = acc1 + rows_v[c, r0 + j, pl.ds(16, 16)] * w
                out_v[pl.ds((r0 + j) * EMB_DIM, 16)] = acc0 * 0.125
                out_v[pl.ds((r0 + j) * EMB_DIM + 16, 16)] = acc1 * 0.125
            return _

        _fori(SUB // 16, racc)
        pltpu.sync_copy(
            out_v, out_hbm.at[pl.ds((base + s * SUB) * EMB_DIM, SUB * EMB_DIM)])
        return carry

    _fori(N_SUB, sblock)


TBLK = 32768
TQ = TBLK // 4
TGRID = (N_ROWS + TBLK - 1) // TBLK
ROWS_PAD = TGRID * TBLK


def _transpose_body(tt_ref, out_ref):
    x = tt_ref[...]
    x4 = jnp.concatenate(
        [x[:, k * TQ:(k + 1) * TQ] for k in range(4)], axis=0)
    out_ref[...] = x4.T


def _relayout_table(table_t):
    return pl.pallas_call(
        _transpose_body,
        grid=(TGRID,),
        in_specs=[pl.BlockSpec((32, TBLK), lambda i: (jnp.int32(0), i))],
        out_specs=pl.BlockSpec((TQ, 128), lambda i: (i, jnp.int32(0))),
        out_shape=jax.ShapeDtypeStruct((TGRID * TQ, 128), jnp.float32),
    )(table_t)


@jax.jit
def _run(x_i32, table, w_flat, coef):
    mesh = plsc.VectorSubcoreMesh(core_axis_name="c", subcore_axis_name="s")
    return pl.kernel(
        _body,
        out_type=jax.ShapeDtypeStruct((B_TOTAL * EMB_DIM,), jnp.float32),
        mesh=mesh,
        compiler_params=pltpu.CompilerParams(use_tc_tiling_on_sc=False),
        scratch_types=[
            pltpu.VMEM((B_PER_W,), jnp.int32),
            pltpu.VMEM((2 * 16,), jnp.uint32),
            pltpu.VMEM((CHUNKS, SUB), jnp.int32),
            pltpu.VMEM((CHUNKS, SUB), jnp.int32),
            pltpu.VMEM((CHUNKS, SUB, EMB_DIM), jnp.float32),
            pltpu.VMEM((CHUNKS, SUB), jnp.float32),
            pltpu.VMEM((SUB * EMB_DIM,), jnp.float32),
            pltpu.SemaphoreType.DMA,
        ],
    )(x_i32, table, w_flat, coef)


@jax.jit
def _entry(x, table, weights, h0_coeffs, h1_coeffs):
    x_i32 = x.astype(jnp.int32)
    w_flat = weights.reshape(-1)
    coef = jnp.concatenate([h0_coeffs[:, 0], h0_coeffs[:, 1],
                            h1_coeffs[:, 0], h1_coeffs[:, 1]]).astype(jnp.uint32)
    t_lin = _relayout_table(table.T).reshape(ROWS_PAD, EMB_DIM)
    return _run(x_i32, t_lin, w_flat, coef).reshape(B_TOTAL, EMB_DIM)


def kernel(x, table, weights, h0_coeffs, h1_coeffs):
    return _entry(x, table, weights, h0_coeffs, h1_coeffs)


# speedup vs baseline: 1.1913x; 1.0075x over previous
import jax
import jax.numpy as jnp
from jax import lax
from jax.experimental import pallas as pl
from jax.experimental.pallas import tpu as pltpu
from jax.experimental.pallas import tpu_sc as plsc

MPRIME = (1 << 31) - 1
N_ROWS = 1000000
EMB_DIM = 32
CHUNKS = 8
B_TOTAL = 16384
NC = 2
NS = 16
NW = NC * NS
B_PER_W = B_TOTAL // NW
SUB = 128
N_SUB = B_PER_W // SUB


RMAGIC = 1125899907


def _magic_rem(s, shift, divisor):
    m0 = jnp.uint32(RMAGIC & 0xFFFF)
    m1 = jnp.uint32(RMAGIC >> 16)
    n0 = s & jnp.uint32(0xFFFF)
    n1 = s >> 16
    lo = n0 * m0
    mid = n0 * m1 + n1 * m0
    u = mid + (lo >> 16)
    hi = n1 * m1 + (u >> 16)
    q = hi >> shift
    return s - q * jnp.uint32(divisor)


def _mersenne_hash(x0, x1, a_lo, a_hi, b_add, out_range):
    m = jnp.uint32(MPRIME)
    p0 = x0 * a_lo
    pm = x0 * a_hi + x1 * a_lo
    p2 = x1 * a_hi
    ra = (p0 >> 31) + (p0 & m)
    ra = (ra >> 31) + (ra & m)
    rb = (pm >> 15) + ((pm & jnp.uint32(0x7FFF)) << 16)
    rb = (rb >> 31) + (rb & m)
    s = ra + rb
    s = (s >> 31) + (s & m)
    s = s + (p2 << 1)
    s = (s >> 31) + (s & m)
    s = s + b_add
    s = (s >> 31) + (s & m)
    s = (s >> 31) + (s & m)
    s = jnp.where(s == m, jnp.uint32(0), s)
    shift = 18 if out_range == N_ROWS else 23
    return plsc.bitcast(_magic_rem(s, shift, out_range), jnp.int32)


def _remap_row(i):
    return ((i & jnp.int32(~(TBLK - 1))) | ((i & jnp.int32(TQ - 1)) << 2)
            | ((i >> (TQ.bit_length() - 1)) & jnp.int32(3)))


def _fori(n, body):
    lax.fori_loop(jnp.int32(0), jnp.int32(n), body, jnp.int32(0))


def _body(x_hbm, table_hbm, w_hbm, coef_hbm, out_hbm,
          xv, coef_v, idx0_v, idx1_v, rows_v, wv_v, out_v, sem):
    wid = lax.axis_index("s") * NC + lax.axis_index("c")
    base = wid * B_PER_W
    pltpu.sync_copy(x_hbm.at[pl.ds(base, B_PER_W)], xv)
    pltpu.sync_copy(coef_hbm, coef_v)

    c01 = coef_v[pl.ds(0, 16)]
    c23 = coef_v[pl.ds(16, 16)]
    cparams = []
    for c in range(CHUNKS):
        a0, b0, a1, b1 = c01[c], c01[8 + c], c23[c], c23[8 + c]
        cparams.append((a0 & jnp.uint32(0xFFFF), a0 >> 16, b0,
                        a1 & jnp.uint32(0xFFFF), a1 >> 16, b1))

    def sblock(s, carry):
      with jax.named_scope("hash"):
        for c in range(CHUNKS):
            a0_lo, a0_hi, b0, a1_lo, a1_hi, b1 = cparams[c]

            def vbody(v, _, c=c, a0_lo=a0_lo, a0_hi=a0_hi, b0=b0,
                      a1_lo=a1_lo, a1_hi=a1_hi, b1=b1):
                for u in range(2):
                    off = s * SUB + v * 32 + u * 16
                    xu = plsc.bitcast(xv[pl.ds(off, 16)], jnp.uint32)
                    x0 = xu & jnp.uint32(0xFFFF)
                    x1 = xu >> 16
                    idx0_v[c, pl.ds(v * 32 + u * 16, 16)] = _remap_row(
                        _mersenne_hash(x0, x1, a0_lo, a0_hi, b0, N_ROWS))
                    idx1_v[c, pl.ds(v * 32 + u * 16, 16)] = _mersenne_hash(
                        x0, x1, a1_lo, a1_hi, b1, N_ROWS * EMB_DIM)
                return _

            _fori(SUB // 32, vbody)

      with jax.named_scope("gather"):
        copies = []
        for c in range(CHUNKS):
            ci = jnp.int32(c)
            copies.append(pltpu.make_async_copy(
                table_hbm.at[idx0_v.at[ci]], rows_v.at[ci], sem))
            copies.append(pltpu.make_async_copy(
                w_hbm.at[idx1_v.at[ci]], wv_v.at[ci], sem))
        for cp in copies:
            cp.start()
        for cp in copies:
            cp.wait()

      with jax.named_scope("acc"):
        def racc(g, _):
            r0 = g * 16
            wvecs = [wv_v[c, pl.ds(r0, 16)] for c in range(CHUNKS)]
            for j in range(16):
                acc0 = jnp.zeros((16,), jnp.float32)
                acc1 = jnp.zeros((16,), jnp.float32)
                for c in range(CHUNKS):
                    w = wvecs[c][j]
                    acc0 = acc0 + rows_v[c, r0 + j, pl.ds(0, 16)] * w
                    acc1 = acc1 + rows_v[c, r0 + j, pl.ds(16, 16)] * w
                out_v[pl.ds((r0 + j) * EMB_DIM, 16)] = acc0 * 0.125
                out_v[pl.ds((r0 + j) * EMB_DIM + 16, 16)] = acc1 * 0.125
            return _

        _fori(SUB // 16, racc)
        pltpu.sync_copy(
            out_v, out_hbm.at[pl.ds((base + s * SUB) * EMB_DIM, SUB * EMB_DIM)])
        return carry

    _fori(N_SUB, sblock)


TBLK = 65536
TQ = TBLK // 4
TGRID = (N_ROWS + TBLK - 1) // TBLK
ROWS_PAD = TGRID * TBLK


def _transpose_body(tt_ref, out_ref):
    x = tt_ref[...]
    x4 = jnp.concatenate(
        [x[:, k * TQ:(k + 1) * TQ] for k in range(4)], axis=0)
    out_ref[...] = x4.T


def _relayout_table(table_t):
    return pl.pallas_call(
        _transpose_body,
        grid=(TGRID,),
        in_specs=[pl.BlockSpec((32, TBLK), lambda i: (jnp.int32(0), i))],
        out_specs=pl.BlockSpec((TQ, 128), lambda i: (i, jnp.int32(0))),
        out_shape=jax.ShapeDtypeStruct((TGRID * TQ, 128), jnp.float32),
    )(table_t)


@jax.jit
def _run(x_i32, table, w_flat, coef):
    mesh = plsc.VectorSubcoreMesh(core_axis_name="c", subcore_axis_name="s")
    return pl.kernel(
        _body,
        out_type=jax.ShapeDtypeStruct((B_TOTAL * EMB_DIM,), jnp.float32),
        mesh=mesh,
        compiler_params=pltpu.CompilerParams(use_tc_tiling_on_sc=False),
        scratch_types=[
            pltpu.VMEM((B_PER_W,), jnp.int32),
            pltpu.VMEM((2 * 16,), jnp.uint32),
            pltpu.VMEM((CHUNKS, SUB), jnp.int32),
            pltpu.VMEM((CHUNKS, SUB), jnp.int32),
            pltpu.VMEM((CHUNKS, SUB, EMB_DIM), jnp.float32),
            pltpu.VMEM((CHUNKS, SUB), jnp.float32),
            pltpu.VMEM((SUB * EMB_DIM,), jnp.float32),
            pltpu.SemaphoreType.DMA,
        ],
    )(x_i32, table, w_flat, coef)


@jax.jit
def _entry(x, table, weights, h0_coeffs, h1_coeffs):
    x_i32 = x.astype(jnp.int32)
    w_flat = weights.reshape(-1)
    coef = jnp.concatenate([h0_coeffs[:, 0], h0_coeffs[:, 1],
                            h1_coeffs[:, 0], h1_coeffs[:, 1]]).astype(jnp.uint32)
    t_lin = _relayout_table(table.T).reshape(ROWS_PAD, EMB_DIM)
    return _run(x_i32, t_lin, w_flat, coef).reshape(B_TOTAL, EMB_DIM)


def kernel(x, table, weights, h0_coeffs, h1_coeffs):
    return _entry(x, table, weights, h0_coeffs, h1_coeffs)


# speedup vs baseline: 1.2520x; 1.0509x over previous
import jax
import jax.numpy as jnp
from jax import lax
from jax.experimental import pallas as pl
from jax.experimental.pallas import tpu as pltpu
from jax.experimental.pallas import tpu_sc as plsc

MPRIME = (1 << 31) - 1
N_ROWS = 1000000
EMB_DIM = 32
CHUNKS = 8
B_TOTAL = 16384
NC = 2
NS = 16
NW = NC * NS
B_PER_W = B_TOTAL // NW
SUB = 128
N_SUB = B_PER_W // SUB


RMAGIC = 1125899907


def _magic_rem(s, shift, divisor):
    m0 = jnp.uint32(RMAGIC & 0xFFFF)
    m1 = jnp.uint32(RMAGIC >> 16)
    n0 = s & jnp.uint32(0xFFFF)
    n1 = s >> 16
    lo = n0 * m0
    mid = n0 * m1 + n1 * m0
    u = mid + (lo >> 16)
    hi = n1 * m1 + (u >> 16)
    q = hi >> shift
    return s - q * jnp.uint32(divisor)


def _mersenne_hash(x0, x1, a_lo, a_hi, b_add, out_range):
    m = jnp.uint32(MPRIME)
    p0 = x0 * a_lo
    pm = x0 * a_hi + x1 * a_lo
    p2 = x1 * a_hi
    ra = (p0 >> 31) + (p0 & m)
    ra = (ra >> 31) + (ra & m)
    rb = (pm >> 15) + ((pm & jnp.uint32(0x7FFF)) << 16)
    rb = (rb >> 31) + (rb & m)
    s = ra + rb
    s = (s >> 31) + (s & m)
    s = s + (p2 << 1)
    s = (s >> 31) + (s & m)
    s = s + b_add
    s = (s >> 31) + (s & m)
    s = (s >> 31) + (s & m)
    s = jnp.where(s == m, jnp.uint32(0), s)
    shift = 18 if out_range == N_ROWS else 23
    return plsc.bitcast(_magic_rem(s, shift, out_range), jnp.int32)


def _remap_row(i):
    return ((i & jnp.int32(~(TBLK - 1))) | ((i & jnp.int32(TQ - 1)) << 2)
            | ((i >> (TQ.bit_length() - 1)) & jnp.int32(3)))


def _fori(n, body):
    lax.fori_loop(jnp.int32(0), jnp.int32(n), body, jnp.int32(0))


def _body(x_hbm, table_hbm, w_hbm, coef_hbm, out_hbm,
          xv, coef_v, idx0_v, idx1_v, rows_v, wv_v, out_v, sem):
    wid = lax.axis_index("s") * NC + lax.axis_index("c")
    base = wid * B_PER_W
    pltpu.sync_copy(x_hbm.at[pl.ds(base, B_PER_W)], xv)
    pltpu.sync_copy(coef_hbm, coef_v)

    c01 = coef_v[pl.ds(0, 16)]
    c23 = coef_v[pl.ds(16, 16)]
    cparams = []
    for c in range(CHUNKS):
        a0, b0, a1, b1 = c01[c], c01[8 + c], c23[c], c23[8 + c]
        cparams.append((a0 & jnp.uint32(0xFFFF), a0 >> 16, b0,
                        a1 & jnp.uint32(0xFFFF), a1 >> 16, b1))

    def sblock(s, carry):
      par = s & 1
      prv = 1 - par

      @pl.when(s < N_SUB)
      def _hash_fire():
        with jax.named_scope("hash"):
          for c in range(CHUNKS):
            a0_lo, a0_hi, b0, a1_lo, a1_hi, b1 = cparams[c]

            def vbody(v, _, c=c, a0_lo=a0_lo, a0_hi=a0_hi, b0=b0,
                      a1_lo=a1_lo, a1_hi=a1_hi, b1=b1):
                for u in range(2):
                    off = s * SUB + v * 32 + u * 16
                    xu = plsc.bitcast(xv[pl.ds(off, 16)], jnp.uint32)
                    x0 = xu & jnp.uint32(0xFFFF)
                    x1 = xu >> 16
                    idx0_v[par, c, pl.ds(v * 32 + u * 16, 16)] = _remap_row(
                        _mersenne_hash(x0, x1, a0_lo, a0_hi, b0, N_ROWS))
                    idx1_v[par, c, pl.ds(v * 32 + u * 16, 16)] = _mersenne_hash(
                        x0, x1, a1_lo, a1_hi, b1, N_ROWS * EMB_DIM)
                return _

            _fori(SUB // 32, vbody)

      @pl.when(s > 0)
      def _drain():
        with jax.named_scope("drain"):
          for c in range(CHUNKS):
            ci = jnp.int32(c)
            pltpu.make_async_copy(
                table_hbm.at[idx0_v.at[prv, ci]], rows_v.at[prv, ci], sem).wait()
            pltpu.make_async_copy(
                w_hbm.at[idx1_v.at[prv, ci]], wv_v.at[prv, ci], sem).wait()

      @pl.when(s < N_SUB)
      def _fire():
        with jax.named_scope("fire"):
          for c in range(CHUNKS):
            ci = jnp.int32(c)
            pltpu.make_async_copy(
                table_hbm.at[idx0_v.at[par, ci]], rows_v.at[par, ci], sem).start()
            pltpu.make_async_copy(
                w_hbm.at[idx1_v.at[par, ci]], wv_v.at[par, ci], sem).start()

      @pl.when(s > 0)
      def _accum():
        with jax.named_scope("acc"):
          def racc(g, _):
            r0 = g * 16
            wvecs = [wv_v[prv, c, pl.ds(r0, 16)] for c in range(CHUNKS)]
            for j in range(16):
                acc0 = jnp.zeros((16,), jnp.float32)
                acc1 = jnp.zeros((16,), jnp.float32)
                for c in range(CHUNKS):
                    w = wvecs[c][j]
                    acc0 = acc0 + rows_v[prv, c, r0 + j, pl.ds(0, 16)] * w
                    acc1 = acc1 + rows_v[prv, c, r0 + j, pl.ds(16, 16)] * w
                out_v[pl.ds((r0 + j) * EMB_DIM, 16)] = acc0 * 0.125
                out_v[pl.ds((r0 + j) * EMB_DIM + 16, 16)] = acc1 * 0.125
            return _

          _fori(SUB // 16, racc)
          pltpu.sync_copy(
              out_v,
              out_hbm.at[pl.ds((base + (s - 1) * SUB) * EMB_DIM, SUB * EMB_DIM)])

      return carry

    _fori(N_SUB + 1, sblock)


TBLK = 65536
TQ = TBLK // 4
TGRID = (N_ROWS + TBLK - 1) // TBLK
ROWS_PAD = TGRID * TBLK


def _transpose_body(tt_ref, out_ref):
    x = tt_ref[...]
    x4 = jnp.concatenate(
        [x[:, k * TQ:(k + 1) * TQ] for k in range(4)], axis=0)
    out_ref[...] = x4.T


def _relayout_table(table_t):
    return pl.pallas_call(
        _transpose_body,
        grid=(TGRID,),
        in_specs=[pl.BlockSpec((32, TBLK), lambda i: (jnp.int32(0), i))],
        out_specs=pl.BlockSpec((TQ, 128), lambda i: (i, jnp.int32(0))),
        out_shape=jax.ShapeDtypeStruct((TGRID * TQ, 128), jnp.float32),
    )(table_t)


@jax.jit
def _run(x_i32, table, w_flat, coef):
    mesh = plsc.VectorSubcoreMesh(core_axis_name="c", subcore_axis_name="s")
    return pl.kernel(
        _body,
        out_type=jax.ShapeDtypeStruct((B_TOTAL * EMB_DIM,), jnp.float32),
        mesh=mesh,
        compiler_params=pltpu.CompilerParams(use_tc_tiling_on_sc=False),
        scratch_types=[
            pltpu.VMEM((B_PER_W,), jnp.int32),
            pltpu.VMEM((2 * 16,), jnp.uint32),
            pltpu.VMEM((2, CHUNKS, SUB), jnp.int32),
            pltpu.VMEM((2, CHUNKS, SUB), jnp.int32),
            pltpu.VMEM((2, CHUNKS, SUB, EMB_DIM), jnp.float32),
            pltpu.VMEM((2, CHUNKS, SUB), jnp.float32),
            pltpu.VMEM((SUB * EMB_DIM,), jnp.float32),
            pltpu.SemaphoreType.DMA,
        ],
    )(x_i32, table, w_flat, coef)


@jax.jit
def _entry(x, table, weights, h0_coeffs, h1_coeffs):
    x_i32 = x.astype(jnp.int32)
    w_flat = weights.reshape(-1)
    coef = jnp.concatenate([h0_coeffs[:, 0], h0_coeffs[:, 1],
                            h1_coeffs[:, 0], h1_coeffs[:, 1]]).astype(jnp.uint32)
    t_lin = _relayout_table(table.T).reshape(ROWS_PAD, EMB_DIM)
    return _run(x_i32, t_lin, w_flat, coef).reshape(B_TOTAL, EMB_DIM)


def kernel(x, table, weights, h0_coeffs, h1_coeffs):
    return _entry(x, table, weights, h0_coeffs, h1_coeffs)
